# baseline (speedup 1.0000x reference)
import functools

import jax
import jax.numpy as jnp
from jax import lax
from jax.experimental import pallas as pl
from jax.experimental.pallas import tpu as pltpu
from jax.experimental.pallas import tpu_sc as plsc

N_CODES = 1024
C_DIM = 256
HW = 1024
N_BATCH = 8


def _vq_kernel(z_ref, cb2x_ref, cbt_ref, cb2_ref, zq_ref, idx_ref, loss_ref):
    zb = z_ref[...]
    cb2x = cb2x_ref[...]
    m2 = lax.dot_general(cb2x, zb, (((1,), (0,)), ((), ())),
                         preferred_element_type=jnp.float32)
    z2 = jnp.sum(zb * zb, axis=0, keepdims=True)
    cb2 = cb2_ref[...]
    d = (z2 + cb2) - m2
    mind = jnp.min(d, axis=0, keepdims=True)
    code_iota = lax.broadcasted_iota(jnp.int32, d.shape, 0)
    idx = jnp.min(jnp.where(d == mind, code_iota, N_CODES), axis=0)
    onehot = jnp.where(code_iota == idx[None, :],
                       jnp.float32(1), jnp.float32(0)).astype(jnp.bfloat16)
    zq_t = lax.dot_general(cbt_ref[...], onehot, (((1,), (0,)), ((), ())),
                           preferred_element_type=jnp.float32)
    zq_ref[...] = zq_t
    idx_ref[...] = idx.reshape(1, HW)
    loss_ref[...] = jnp.broadcast_to(jnp.sum(mind), (1, 128))


_vq_call = pl.pallas_call(
    _vq_kernel,
    grid=(N_BATCH,),
    in_specs=[
        pl.BlockSpec((None, C_DIM, HW), lambda i: (i, 0, 0)),
        pl.BlockSpec((N_CODES, C_DIM), lambda i: (0, 0)),
        pl.BlockSpec((C_DIM, N_CODES), lambda i: (0, 0)),
        pl.BlockSpec((N_CODES, 1), lambda i: (0, 0)),
    ],
    out_specs=[
        pl.BlockSpec((None, C_DIM, HW), lambda i: (i, 0, 0)),
        pl.BlockSpec((None, 1, HW), lambda i: (i, 0, 0)),
        pl.BlockSpec((None, 1, 128), lambda i: (i, 0, 0)),
    ],
    out_shape=[
        jax.ShapeDtypeStruct((N_BATCH, C_DIM, HW), jnp.float32),
        jax.ShapeDtypeStruct((N_BATCH, 1, HW), jnp.int32),
        jax.ShapeDtypeStruct((N_BATCH, 1, 128), jnp.float32),
    ],
)


def kernel(z, codebook):
    B, C, H, W = z.shape
    zb = z.reshape(B, C_DIM, HW)
    cbt = jnp.transpose(codebook).astype(jnp.bfloat16)
    cb2x = codebook + codebook
    cb2 = jnp.sum(codebook * codebook, axis=1, keepdims=True)
    zq, idx8, loss_part = _vq_call(zb, cb2x, cbt, cb2)
    z_q_out = zq.reshape(B, C, H, W)
    codebook_loss = jnp.sum(loss_part[:, 0, 0]) / (B * C * H * W)
    cls_loss = jnp.zeros((), jnp.float32)
    indices_out = idx8.reshape(B, 1, H, W)
    return (z_q_out, codebook_loss, cls_loss, indices_out)


# speedup vs baseline: 1.0977x; 1.0977x over previous
import functools

import jax
import jax.numpy as jnp
from jax import lax
from jax.experimental import pallas as pl
from jax.experimental.pallas import tpu as pltpu
from jax.experimental.pallas import tpu_sc as plsc

N_CODES = 1024
C_DIM = 256
HW = 1024
N_BATCH = 8


def _vq_kernel(z_ref, cb_ref, zq_ref, idx_ref, loss_ref):
    zb = z_ref[...]
    cb = cb_ref[...]
    m = lax.dot_general(cb, zb, (((1,), (0,)), ((), ())),
                        preferred_element_type=jnp.float32)
    z2 = jnp.sum(zb * zb, axis=0, keepdims=True)
    cb2 = jnp.sum(cb * cb, axis=1, keepdims=True)
    d = (z2 + cb2) - 2.0 * m
    mind = jnp.min(d, axis=0, keepdims=True)
    code_iota = lax.broadcasted_iota(jnp.int32, d.shape, 0)
    idx = jnp.min(jnp.where(d == mind, code_iota, N_CODES), axis=0)
    onehot = jnp.where(code_iota == idx[None, :],
                       jnp.float32(1), jnp.float32(0))
    zq_t = lax.dot_general(cb, onehot, (((0,), (0,)), ((), ())),
                           preferred_element_type=jnp.float32)
    zq_ref[...] = zq_t
    idx_ref[...] = idx.reshape(1, HW)
    loss_ref[...] = jnp.broadcast_to(jnp.sum(mind), (1, 128))


_vq_call = pl.pallas_call(
    _vq_kernel,
    grid=(N_BATCH,),
    in_specs=[
        pl.BlockSpec((None, C_DIM, HW), lambda i: (i, 0, 0)),
        pl.BlockSpec((N_CODES, C_DIM), lambda i: (0, 0)),
    ],
    out_specs=[
        pl.BlockSpec((None, C_DIM, HW), lambda i: (i, 0, 0)),
        pl.BlockSpec((None, 1, HW), lambda i: (i, 0, 0)),
        pl.BlockSpec((None, 1, 128), lambda i: (i, 0, 0)),
    ],
    out_shape=[
        jax.ShapeDtypeStruct((N_BATCH, C_DIM, HW), jnp.float32),
        jax.ShapeDtypeStruct((N_BATCH, 1, HW), jnp.int32),
        jax.ShapeDtypeStruct((N_BATCH, 1, 128), jnp.float32),
    ],
)


def kernel(z, codebook):
    B, C, H, W = z.shape
    zb = z.reshape(B, C_DIM, HW)
    zq, idx8, loss_part = _vq_call(zb, codebook)
    z_q_out = zq.reshape(B, C, H, W)
    codebook_loss = jnp.sum(loss_part[:, 0, 0]) / (B * C * H * W)
    cls_loss = jnp.zeros((), jnp.float32)
    indices_out = idx8.reshape(B, 1, H, W)
    return (z_q_out, codebook_loss, cls_loss, indices_out)
